# baseline (device time: 264394 ns/iter reference)
import functools

import jax
import jax.numpy as jnp
from jax import lax
from jax.experimental import pallas as pl
from jax.experimental.pallas import tpu as pltpu

N_DEV = 4
F_TILE = 128
N_LAYERS = 3


def _all_peer_barrier(my):
    barrier_sem = pltpu.get_barrier_semaphore()
    for k in range(1, N_DEV):
        peer = lax.rem(my + k, N_DEV)
        pl.semaphore_signal(
            barrier_sem, inc=1,
            device_id=(peer,), device_id_type=pl.DeviceIdType.MESH,
        )
    pl.semaphore_wait(barrier_sem, N_DEV - 1)


def _exit_barrier(my):
    @functools.partial(pl.run_scoped, sem=pltpu.SemaphoreType.REGULAR)
    def _(sem):
        for k in range(1, N_DEV):
            peer = lax.rem(my + k, N_DEV)
            pl.semaphore_signal(
                sem, inc=1,
                device_id=(peer,), device_id_type=pl.DeviceIdType.MESH,
            )
        pl.semaphore_wait(sem, N_DEV - 1)


def kernel(x, Win0, Wout0, Win1, Wout1, Win2, Wout2):
    m_per, d = x.shape
    b = N_DEV * m_per
    f = Win0.shape[1]
    n_t = f // F_TILE

    def body(x_ref, win0, wout0, win1, wout1, win2, wout2, out_ref,
             xbuf, pbuf, rs_buf,
             agx_send, agx_recv, rs_send, rs_recv, ag_send, ag_recv):
        my = lax.axis_index("i")
        l = pl.program_id(0)
        t = pl.program_id(1)

        @pl.when((l == 0) & (t == 0))
        def _():
            _all_peer_barrier(my)
            xbuf[pl.ds(my * m_per, m_per), :] = x_ref[...]
            sends = []
            for k in range(1, N_DEV):
                peer = lax.rem(my + k, N_DEV)
                rdma = pltpu.make_async_remote_copy(
                    src_ref=xbuf.at[pl.ds(my * m_per, m_per), :],
                    dst_ref=xbuf.at[pl.ds(my * m_per, m_per), :],
                    send_sem=agx_send.at[k - 1],
                    recv_sem=agx_recv.at[3 - k],
                    device_id=(peer,),
                    device_id_type=pl.DeviceIdType.MESH,
                )
                rdma.start()
                sends.append(rdma)
            for s in range(N_DEV - 1):
                peer = lax.rem(my + 3 - s, N_DEV)
                recv = pltpu.make_async_remote_copy(
                    src_ref=xbuf.at[pl.ds(peer * m_per, m_per), :],
                    dst_ref=xbuf.at[pl.ds(peer * m_per, m_per), :],
                    send_sem=agx_send.at[0],
                    recv_sem=agx_recv.at[s],
                    device_id=(peer,),
                    device_id_type=pl.DeviceIdType.MESH,
                )
                recv.wait_recv()
            for rdma in sends:
                rdma.wait_send()

        win = [win0, win1, win2]
        wout = [wout0, wout1, wout2]
        for li in range(N_LAYERS):
            @pl.when(l == li)
            def _(li=li):
                h = jnp.dot(xbuf[...], win[li][...],
                            preferred_element_type=jnp.float32)
                h = jnp.maximum(h, 0.0)
                p = jnp.dot(h, wout[li][...],
                            preferred_element_type=jnp.float32)

                @pl.when(t == 0)
                def _():
                    pbuf[...] = p

                @pl.when(t > 0)
                def _():
                    pbuf[...] += p

        @pl.when(t == n_t - 1)
        def _():
            rs_sends = []
            for k in range(1, N_DEV):
                peer = lax.rem(my + k, N_DEV)
                rdma = pltpu.make_async_remote_copy(
                    src_ref=pbuf.at[pl.ds(peer * m_per, m_per), :],
                    dst_ref=rs_buf.at[l, 3 - k],
                    send_sem=rs_send.at[l, k - 1],
                    recv_sem=rs_recv.at[l, 3 - k],
                    device_id=(peer,),
                    device_id_type=pl.DeviceIdType.MESH,
                )
                rdma.start()
                rs_sends.append(rdma)
            for s in range(N_DEV - 1):
                recv = pltpu.make_async_remote_copy(
                    src_ref=rs_buf.at[l, s],
                    dst_ref=rs_buf.at[l, s],
                    send_sem=rs_send.at[l, 0],
                    recv_sem=rs_recv.at[l, s],
                    device_id=(my,),
                    device_id_type=pl.DeviceIdType.MESH,
                )
                recv.wait_recv()

            reduced = (pbuf[pl.ds(my * m_per, m_per), :]
                       + rs_buf[l, 0] + rs_buf[l, 1] + rs_buf[l, 2])

            def do_ag(target):
                target[pl.ds(my * m_per, m_per), :] = reduced
                ag_sends = []
                for k in range(1, N_DEV):
                    peer = lax.rem(my + k, N_DEV)
                    rdma = pltpu.make_async_remote_copy(
                        src_ref=target.at[pl.ds(my * m_per, m_per), :],
                        dst_ref=target.at[pl.ds(my * m_per, m_per), :],
                        send_sem=ag_send.at[l, k - 1],
                        recv_sem=ag_recv.at[l, 3 - k],
                        device_id=(peer,),
                        device_id_type=pl.DeviceIdType.MESH,
                    )
                    rdma.start()
                    ag_sends.append(rdma)
                for s in range(N_DEV - 1):
                    peer = lax.rem(my + 3 - s, N_DEV)
                    recv = pltpu.make_async_remote_copy(
                        src_ref=target.at[pl.ds(peer * m_per, m_per), :],
                        dst_ref=target.at[pl.ds(peer * m_per, m_per), :],
                        send_sem=ag_send.at[l, 0],
                        recv_sem=ag_recv.at[l, s],
                        device_id=(peer,),
                        device_id_type=pl.DeviceIdType.MESH,
                    )
                    recv.wait_recv()
                for rdma in ag_sends:
                    rdma.wait_send()

            @pl.when(l < N_LAYERS - 1)
            def _():
                do_ag(xbuf)

            @pl.when(l == N_LAYERS - 1)
            def _():
                do_ag(out_ref)
                _exit_barrier(my)

            for rdma in rs_sends:
                rdma.wait_send()

    frozen0 = n_t - 1

    def win_map(li):
        def m(l, t):
            if li == 0:
                return (0, jnp.where(l == 0, t, frozen0))
            if li == 1:
                return (0, jnp.where(l < 1, 0, jnp.where(l == 1, t, frozen0)))
            return (0, jnp.where(l < 2, 0, t))
        return m

    def wout_map(li):
        def m(l, t):
            if li == 0:
                return (jnp.where(l == 0, t, frozen0), 0)
            if li == 1:
                return (jnp.where(l < 1, 0, jnp.where(l == 1, t, frozen0)), 0)
            return (jnp.where(l < 2, 0, t), 0)
        return m

    return pl.pallas_call(
        body,
        grid=(N_LAYERS, n_t),
        out_shape=jax.ShapeDtypeStruct((b, d), jnp.float32),
        in_specs=[
            pl.BlockSpec((m_per, d), lambda l, t: (0, 0)),
            pl.BlockSpec((d, F_TILE), win_map(0)),
            pl.BlockSpec((F_TILE, d), wout_map(0)),
            pl.BlockSpec((d, F_TILE), win_map(1)),
            pl.BlockSpec((F_TILE, d), wout_map(1)),
            pl.BlockSpec((d, F_TILE), win_map(2)),
            pl.BlockSpec((F_TILE, d), wout_map(2)),
        ],
        out_specs=pl.BlockSpec((b, d), lambda l, t: (0, 0)),
        scratch_shapes=[
            pltpu.VMEM((b, d), jnp.float32),
            pltpu.VMEM((b, d), jnp.float32),
            pltpu.VMEM((N_LAYERS, N_DEV - 1, m_per, d), jnp.float32),
            pltpu.SemaphoreType.DMA((N_DEV - 1,)),
            pltpu.SemaphoreType.DMA((N_DEV - 1,)),
            pltpu.SemaphoreType.DMA((N_LAYERS, N_DEV - 1)),
            pltpu.SemaphoreType.DMA((N_LAYERS, N_DEV - 1)),
            pltpu.SemaphoreType.DMA((N_LAYERS, N_DEV - 1)),
            pltpu.SemaphoreType.DMA((N_LAYERS, N_DEV - 1)),
        ],
        compiler_params=pltpu.CompilerParams(
            collective_id=0, vmem_limit_bytes=48 * 1024 * 1024,
        ),
    )(x, Win0, Wout0, Win1, Wout1, Win2, Wout2)
